# Initial kernel scaffold; baseline (speedup 1.0000x reference)
import jax
import jax.numpy as jnp
from jax.experimental import pallas as pl


def kernel(src, seg, edge_index, src_table, seg_table, w, q1_W, q1_b, q2_W, q2_b, conv1_W, conv1_b, conv2_W, conv2_b, lin_W, lin_b):
    raise NotImplementedError("write your pallas kernel here")


# speedup vs baseline: 6.5887x; 6.5887x over previous
import functools

import jax
import jax.numpy as jnp
from jax import lax
from jax.experimental import pallas as pl
from jax.experimental.pallas import tpu as pltpu
from jax.experimental.pallas import tpu_sc as plsc

N = 10000
NPAD = 10240
E = 320000
L = 32
D = 128
H = 256

NC = 2
NS = 16
NW = NC * NS
EW = E // NW
CH = 128
NFULL = EW // CH
REM = EW - NFULL * CH
ROWS_PER_TILE = NPAD // NS


def _get_mesh():
    return plsc.VectorSubcoreMesh(core_axis_name="c", subcore_axis_name="s")


def _sc_gather(src_table, src_flat):
    @functools.partial(
        pl.kernel,
        mesh=_get_mesh(),
        out_type=jax.ShapeDtypeStruct((NPAD * L, D), jnp.float32),
        scratch_types=[
            pltpu.VMEM((CH,), jnp.int32),
            pltpu.VMEM((CH, D), jnp.float32),
            pltpu.VMEM((REM,), jnp.int32),
            pltpu.VMEM((REM, D), jnp.float32),
        ],
    )
    def k(table_h, idx_h, emb_h, idx_v, rows_v, idxr_v, rowsr_v):
        c = lax.axis_index("c")
        s = lax.axis_index("s")
        wid = s * NC + c
        base0 = wid * EW

        @pl.loop(0, NFULL)
        def _(g):
            base = base0 + g * CH
            pltpu.sync_copy(idx_h.at[pl.ds(base, CH)], idx_v)
            pltpu.sync_copy(table_h.at[idx_v], rows_v)
            pltpu.sync_copy(rows_v, emb_h.at[pl.ds(base, CH)])

        base = base0 + NFULL * CH
        pltpu.sync_copy(idx_h.at[pl.ds(base, REM)], idxr_v)
        pltpu.sync_copy(table_h.at[idxr_v], rowsr_v)
        pltpu.sync_copy(rowsr_v, emb_h.at[pl.ds(base, REM)])

    return k(src_table, src_flat)


def _sc_deg(d_idx, zeros_rows, ones_rows):
    @functools.partial(
        pl.kernel,
        mesh=_get_mesh(),
        out_type=jax.ShapeDtypeStruct((NC * NPAD, D), jnp.float32),
        scratch_types=[
            pltpu.VMEM((CH,), jnp.int32),
            pltpu.VMEM((CH, D), jnp.float32),
            pltpu.VMEM((REM,), jnp.int32),
            pltpu.VMEM((CH, D), jnp.float32),
            pltpu.VMEM((REM, D), jnp.float32),
            pltpu.VMEM_SHARED((NPAD, D), jnp.float32),
        ],
    )
    def k(didx_h, zrows_h, ones_h, deg_h,
          idx_v, rows_v, idxr_v, ones_v, onesr_v, degacc):
        c = lax.axis_index("c")
        s = lax.axis_index("s")
        wid = s * NC + c
        pltpu.sync_copy(zrows_h, rows_v)
        pltpu.sync_copy(ones_h, ones_v)
        pltpu.sync_copy(ones_h.at[pl.ds(0, REM)], onesr_v)

        def fill_iota(base_r):
            for q in range(CH // 16):
                idx_v[pl.ds(q * 16, 16)] = (
                    lax.iota(jnp.int32, 16) + (base_r + q * 16))

        for j in range(ROWS_PER_TILE // CH):
            fill_iota(s * ROWS_PER_TILE + j * CH)
            pltpu.sync_copy(rows_v, degacc.at[idx_v])
        plsc.subcore_barrier()

        base0 = wid * EW

        @pl.loop(0, NFULL)
        def _(g):
            base = base0 + g * CH
            pltpu.sync_copy(didx_h.at[pl.ds(base, CH)], idx_v)
            pltpu.sync_copy(ones_v, degacc.at[idx_v], add=True)

        base = base0 + NFULL * CH
        pltpu.sync_copy(didx_h.at[pl.ds(base, REM)], idxr_v)
        pltpu.sync_copy(onesr_v, degacc.at[idxr_v], add=True)

        plsc.subcore_barrier()
        for j in range(ROWS_PER_TILE // CH):
            base_r = s * ROWS_PER_TILE + j * CH
            fill_iota(base_r)
            pltpu.sync_copy(degacc.at[idx_v], rows_v)
            pltpu.sync_copy(rows_v, deg_h.at[pl.ds(c * NPAD + base_r, CH)])

    return k(d_idx, zeros_rows, ones_rows)


def _sc_conv(ys, s_idx, d_idx, zeros_rows):
    @functools.partial(
        pl.kernel,
        mesh=_get_mesh(),
        out_type=jax.ShapeDtypeStruct((NC * NPAD, D), jnp.float32),
        scratch_types=[
            pltpu.VMEM((CH,), jnp.int32),
            pltpu.VMEM((CH,), jnp.int32),
            pltpu.VMEM((CH, D), jnp.float32),
            pltpu.VMEM((REM,), jnp.int32),
            pltpu.VMEM((REM,), jnp.int32),
            pltpu.VMEM((REM, D), jnp.float32),
            pltpu.VMEM((CH, D), jnp.float32),
            pltpu.VMEM_SHARED((NPAD, D), jnp.float32),
        ],
    )
    def k(ys_h, sidx_h, didx_h, zrows_h, acc_h,
          sidx_v, didx_v, rows_v, sidxr_v, didxr_v, rowsr_v, zeros_v, accs):
        c = lax.axis_index("c")
        s = lax.axis_index("s")
        wid = s * NC + c
        pltpu.sync_copy(zrows_h, zeros_v)

        def fill_iota(base_r):
            for q in range(CH // 16):
                sidx_v[pl.ds(q * 16, 16)] = (
                    lax.iota(jnp.int32, 16) + (base_r + q * 16))

        for j in range(ROWS_PER_TILE // CH):
            fill_iota(s * ROWS_PER_TILE + j * CH)
            pltpu.sync_copy(zeros_v, accs.at[sidx_v])
        plsc.subcore_barrier()

        base0 = wid * EW

        @pl.loop(0, NFULL)
        def _(g):
            base = base0 + g * CH
            pltpu.sync_copy(sidx_h.at[pl.ds(base, CH)], sidx_v)
            pltpu.sync_copy(didx_h.at[pl.ds(base, CH)], didx_v)
            pltpu.sync_copy(ys_h.at[sidx_v], rows_v)
            pltpu.sync_copy(rows_v, accs.at[didx_v], add=True)

        base = base0 + NFULL * CH
        pltpu.sync_copy(sidx_h.at[pl.ds(base, REM)], sidxr_v)
        pltpu.sync_copy(didx_h.at[pl.ds(base, REM)], didxr_v)
        pltpu.sync_copy(ys_h.at[sidxr_v], rowsr_v)
        pltpu.sync_copy(rowsr_v, accs.at[didxr_v], add=True)

        plsc.subcore_barrier()
        for z in range(ROWS_PER_TILE // CH):
            off = s * ROWS_PER_TILE + z * CH
            fill_iota(off)
            pltpu.sync_copy(accs.at[sidx_v], rows_v)
            pltpu.sync_copy(rows_v, acc_h.at[pl.ds(c * NPAD + off, CH)])

    return k(ys, s_idx, d_idx, zeros_rows)


BN = 256
GRID = NPAD // BN

_DOT = functools.partial(jnp.dot, preferred_element_type=jnp.float32,
                         precision=lax.Precision.HIGHEST)


def _tc1_body(emb_r, seg_r, st_r, w_r, deg_r,
              q1w_r, q1b_r, q2w_r, q2b_r, c1w_r, y1s_r, dinv_r):
    A = emb_r[...].reshape(BN, L, 4, 32)
    wv = w_r[0, :]
    sq = jnp.sum(A * A, axis=(1, 2))
    den = jnp.maximum(jnp.abs(wv)[None, :] * jnp.sqrt(sq), 1e-12)
    cs = wv[None, :] / den
    f_src = jnp.sum(A * cs[:, None, None, :], axis=3).reshape(BN, D)

    seg = seg_r[...]
    T4 = st_r[...].reshape(3, 4, 32)
    Sv = jnp.sum(T4 * T4, axis=1)
    is_v = [(seg == v).astype(jnp.float32) for v in range(3)]
    cnt = [jnp.sum(m, axis=1) for m in is_v]
    nsg = sum(cnt[v][:, None] * Sv[v][None, :] for v in range(3))
    cg = 1.0 / jnp.maximum(jnp.sqrt(nsg), 1e-12)
    P = [jnp.sum(cg[:, None, :] * T4[v][None, :, :], axis=2) for v in range(3)]
    f_ge = sum(is_v[v][:, :, None] * P[v][:, None, :] for v in range(3))
    f = f_src + f_ge.reshape(BN, D)

    h = jnp.maximum(_DOT(f, q1w_r[...]) + q1b_r[0, :][None, :], 0.0)
    x = _DOT(h, q2w_r[...]) + q2b_r[0, :][None, :]
    y1 = _DOT(x, c1w_r[...])

    deg = jnp.sum(deg_r[...], axis=(0, 2)) + 1.0
    dinv = lax.rsqrt(deg)
    y1s_r[...] = y1 * dinv[:, None]
    dinv_r[...] = jnp.broadcast_to(dinv[:, None], (BN, D))


def _tc1(emb2, seg_p, seg_table, w2, deg3, q1_W, q1_b, q2_W, q2_b, conv1_W):
    full = lambda shp: pl.BlockSpec(shp, lambda i: tuple(0 for _ in shp))
    return pl.pallas_call(
        _tc1_body,
        grid=(GRID,),
        in_specs=[
            pl.BlockSpec((BN, L * D), lambda i: (i, 0)),
            pl.BlockSpec((BN, L), lambda i: (i, 0)),
            full((3, D)),
            full((1, L)),
            pl.BlockSpec((NC, BN, D), lambda i: (0, i, 0)),
            full((D, H)),
            full((1, H)),
            full((H, D)),
            full((1, D)),
            full((D, D)),
        ],
        out_specs=[
            pl.BlockSpec((BN, D), lambda i: (i, 0)),
            pl.BlockSpec((BN, D), lambda i: (i, 0)),
        ],
        out_shape=[
            jax.ShapeDtypeStruct((NPAD, D), jnp.float32),
            jax.ShapeDtypeStruct((NPAD, D), jnp.float32),
        ],
    )(emb2, seg_p, seg_table, w2, deg3, q1_W, q1_b, q2_W, q2_b, conv1_W)


def _tc2_body(acc_r, ys_r, dinv_r, b_r, w_r, out_r):
    agg = acc_r[0, :, :] + acc_r[1, :, :]
    pre = dinv_r[...] * (agg + ys_r[...]) + b_r[0, :][None, :]
    h = jnp.maximum(pre, 0.0)
    out_r[...] = _DOT(h, w_r[...]) * dinv_r[...]


def _tc2(acc3, y1s, dinv, conv1_b, conv2_W):
    full = lambda shp: pl.BlockSpec(shp, lambda i: tuple(0 for _ in shp))
    return pl.pallas_call(
        _tc2_body,
        grid=(GRID,),
        in_specs=[
            pl.BlockSpec((NC, BN, D), lambda i: (0, i, 0)),
            pl.BlockSpec((BN, D), lambda i: (i, 0)),
            pl.BlockSpec((BN, D), lambda i: (i, 0)),
            full((1, D)),
            full((D, D)),
        ],
        out_specs=pl.BlockSpec((BN, D), lambda i: (i, 0)),
        out_shape=jax.ShapeDtypeStruct((NPAD, D), jnp.float32),
    )(acc3, y1s, dinv, conv1_b, conv2_W)


def _tc3_body(acc_r, ys_r, dinv_r, b_r, w_r, lb_r, out_r):
    agg = acc_r[0, :, :] + acc_r[1, :, :]
    pre = dinv_r[...] * (agg + ys_r[...]) + b_r[0, :][None, :]
    h = jnp.maximum(pre, 0.0)
    out_r[...] = _DOT(h, w_r[...]) + lb_r[0, :][None, :]


def _tc3(acc3, y2s, dinv, conv2_b, lin_W, lin_b):
    full = lambda shp: pl.BlockSpec(shp, lambda i: tuple(0 for _ in shp))
    return pl.pallas_call(
        _tc3_body,
        grid=(GRID,),
        in_specs=[
            pl.BlockSpec((NC, BN, D), lambda i: (0, i, 0)),
            pl.BlockSpec((BN, D), lambda i: (i, 0)),
            pl.BlockSpec((BN, D), lambda i: (i, 0)),
            full((1, D)),
            full((D, D)),
            full((1, D)),
        ],
        out_specs=pl.BlockSpec((BN, D), lambda i: (i, 0)),
        out_shape=jax.ShapeDtypeStruct((NPAD, D), jnp.float32),
    )(acc3, y2s, dinv, conv2_b, lin_W, lin_b)


def kernel(src, seg, edge_index, src_table, seg_table, w,
           q1_W, q1_b, q2_W, q2_b, conv1_W, conv1_b,
           conv2_W, conv2_b, lin_W, lin_b):
    src_flat = src.reshape(-1).astype(jnp.int32)
    s_idx = edge_index[0].astype(jnp.int32)
    d_idx = edge_index[1].astype(jnp.int32)

    zeros_rows = jnp.zeros((CH, D), jnp.float32)
    ones_rows = jnp.full((CH, D), 1.0 / D, jnp.float32)

    emb = _sc_gather(src_table, src_flat)
    degw = _sc_deg(d_idx, zeros_rows, ones_rows)
    emb2 = emb.reshape(NPAD, L * D)
    seg_p = jnp.pad(seg.astype(jnp.int32), ((0, NPAD - N), (0, 0)))
    deg3 = degw.reshape(NC, NPAD, D)

    y1s, dinv = _tc1(emb2, seg_p, seg_table, w.reshape(1, L), deg3,
                     q1_W, q1_b.reshape(1, H), q2_W, q2_b.reshape(1, D), conv1_W)

    acc1 = _sc_conv(y1s, s_idx, d_idx, zeros_rows).reshape(NC, NPAD, D)
    y2s = _tc2(acc1, y1s, dinv, conv1_b.reshape(1, D), conv2_W)

    acc2 = _sc_conv(y2s, s_idx, d_idx, zeros_rows).reshape(NC, NPAD, D)
    out = _tc3(acc2, y2s, dinv, conv2_b.reshape(1, D), lin_W, lin_b.reshape(1, D))
    return out[:N]
